# baseline (speedup 1.0000x reference)
import functools

import jax
import jax.numpy as jnp
from jax import lax
from jax.experimental import pallas as pl
from jax.experimental.pallas import tpu as pltpu
from jax.experimental.pallas import tpu_sc as plsc

_INFO = plsc.get_sparse_core_info()
_NC, _NS = _INFO.num_cores, _INFO.num_subcores
_NW = _NC * _NS
_CH = 128


def _make_sc_gather(B, D):
    bpw = B // _NW
    nch = bpw // _CH
    rpc = B // _CH
    mesh = plsc.VectorSubcoreMesh(core_axis_name="c", subcore_axis_name="s")

    @functools.partial(
        pl.kernel,
        mesh=mesh,
        out_type=[
            jax.ShapeDtypeStruct((B, D), jnp.float32),
            jax.ShapeDtypeStruct((B, D), jnp.float32),
        ],
        scratch_types=[
            pltpu.VMEM((nch, _CH), jnp.int32),
            pltpu.VMEM((nch, _CH), jnp.int32),
            pltpu.VMEM((bpw, D), jnp.float32),
            pltpu.VMEM((bpw, D), jnp.float32),
            pltpu.SemaphoreType.DMA,
        ],
        compiler_params=pltpu.CompilerParams(use_tc_tiling_on_sc=False),
    )
    def gather(uids2d, iids2d, utab, itab, u_out, i_out,
               uidx, iidx, urows, irows, sem):
        wid = lax.axis_index("s") * _NC + lax.axis_index("c")
        base = wid * bpw
        crow = wid * nch
        pltpu.sync_copy(uids2d.at[pl.ds(crow, nch)], uidx)
        pltpu.sync_copy(iids2d.at[pl.ds(crow, nch)], iidx)
        copies = []
        for j in range(nch):
            copies.append(pltpu.async_copy(
                utab.at[uidx.at[j]], urows.at[pl.ds(j * _CH, _CH)], sem))
            copies.append(pltpu.async_copy(
                itab.at[iidx.at[j]], irows.at[pl.ds(j * _CH, _CH)], sem))
        for c in copies:
            c.wait()
        pltpu.sync_copy(urows, u_out.at[pl.ds(base, bpw)])
        pltpu.sync_copy(irows, i_out.at[pl.ds(base, bpw)])

    return gather


def _mlp_body(xu, xi, w1a, w1b, b1, w2, b2, w3, b3, out):
    h = jnp.dot(xu[...], w1a[...], preferred_element_type=jnp.float32)
    h = h + jnp.dot(xi[...], w1b[...], preferred_element_type=jnp.float32)
    h = jnp.maximum(h + b1[...], 0.0)
    h = jnp.maximum(
        jnp.dot(h, w2[...], preferred_element_type=jnp.float32) + b2[...], 0.0)
    out[...] = jnp.dot(h, w3[...], preferred_element_type=jnp.float32) + b3[...]


def _mlp_tc(xu, xi, w1a, w1b, b1, w2, b2, w3, b3, blk=2048):
    B, D = xu.shape
    H1 = w1a.shape[1]
    H2 = w2.shape[1]
    grid = (B // blk,)
    full = lambda shape: pl.BlockSpec(shape, lambda i: (0, 0))
    return pl.pallas_call(
        _mlp_body,
        grid=grid,
        in_specs=[
            pl.BlockSpec((blk, D), lambda i: (i, 0)),
            pl.BlockSpec((blk, D), lambda i: (i, 0)),
            full((D, H1)), full((D, H1)), full((1, H1)),
            full((H1, H2)), full((1, H2)),
            full((H2, 1)), full((1, 1)),
        ],
        out_specs=pl.BlockSpec((blk, 1), lambda i: (i, 0)),
        out_shape=jax.ShapeDtypeStruct((B, 1), jnp.float32),
    )(xu, xi, w1a, w1b, b1, w2, b2, w3, b3)


def kernel(user_ids, item_ids, user_table, item_table, W1, b1, W2, b2, W3, b3):
    B = user_ids.shape[0]
    D = user_table.shape[1]
    uids2d = user_ids.astype(jnp.int32).reshape(B // _CH, _CH)
    iids2d = item_ids.astype(jnp.int32).reshape(B // _CH, _CH)
    u_emb, i_emb = _make_sc_gather(B, D)(uids2d, iids2d, user_table, item_table)
    out = _mlp_tc(u_emb, i_emb, W1[:D], W1[D:], b1.reshape(1, -1),
                  W2, b2.reshape(1, -1), W3, b3.reshape(1, -1))
    return jnp.squeeze(out, axis=1)


# speedup vs baseline: 1.4949x; 1.4949x over previous
import functools

import jax
import jax.numpy as jnp
from jax import lax
from jax.experimental import pallas as pl
from jax.experimental.pallas import tpu as pltpu
from jax.experimental.pallas import tpu_sc as plsc

_INFO = plsc.get_sparse_core_info()
_NC, _NS = _INFO.num_cores, _INFO.num_subcores
_NW = _NC * _NS
_CH = 128


def _make_sc_gather(B, D):
    bpw = B // _NW
    mesh = plsc.VectorSubcoreMesh(core_axis_name="c", subcore_axis_name="s")

    hpw = bpw // 2

    @functools.partial(
        pl.kernel,
        mesh=mesh,
        out_type=[
            jax.ShapeDtypeStruct((B, D), jnp.float32),
            jax.ShapeDtypeStruct((B, D), jnp.float32),
        ],
    scratch_types=[
            pltpu.VMEM((2, bpw), jnp.int32),
            pltpu.VMEM((hpw, D), jnp.float32),
            pltpu.VMEM((hpw, D), jnp.float32),
            pltpu.SemaphoreType.DMA,
            pltpu.SemaphoreType.DMA,
        ],
        compiler_params=pltpu.CompilerParams(
            use_tc_tiling_on_sc=True, needs_layout_passes=False),
    )
    def gather(uids, iids, utab, itab, u_out, i_out,
               ids_vm, urows, irows, usem, isem):
        wid = lax.axis_index("s") * _NC + lax.axis_index("c")
        base = wid * bpw
        pltpu.sync_copy(uids.at[pl.ds(base, bpw)], ids_vm.at[0])
        pltpu.sync_copy(iids.at[pl.ds(base, bpw)], ids_vm.at[1])
        lane = lax.iota(jnp.int32, 16)

        for h in range(2):
            off = h * hpw

            def issue(g, _):
                uvec = ids_vm[0, pl.ds(off + g * 16, 16)]
                ivec = ids_vm[1, pl.ds(off + g * 16, 16)]
                for l in range(16):
                    uid = jnp.sum(jnp.where(lane == l, uvec, 0))
                    iid = jnp.sum(jnp.where(lane == l, ivec, 0))
                    j = g * 16 + l
                    pltpu.make_async_copy(
                        utab.at[pl.ds(uid, 1)], urows.at[pl.ds(j, 1)],
                        usem).start()
                    pltpu.make_async_copy(
                        itab.at[pl.ds(iid, 1)], irows.at[pl.ds(j, 1)],
                        isem).start()
                return 0

            lax.fori_loop(0, hpw // 16, issue, 0)
            pltpu.make_async_copy(utab.at[pl.ds(0, hpw)], urows, usem).wait()
            pltpu.make_async_copy(itab.at[pl.ds(0, hpw)], irows, isem).wait()
            pltpu.sync_copy(urows, u_out.at[pl.ds(base + off, hpw)])
            pltpu.sync_copy(irows, i_out.at[pl.ds(base + off, hpw)])

    return gather


def _mlp_body(xu, xi, w1a, w1b, b1, w2, b2, w3, b3, out):
    h = jnp.dot(xu[...], w1a[...], preferred_element_type=jnp.float32)
    h = h + jnp.dot(xi[...], w1b[...], preferred_element_type=jnp.float32)
    h = jnp.maximum(h + b1[...], 0.0)
    h = jnp.maximum(
        jnp.dot(h, w2[...], preferred_element_type=jnp.float32) + b2[...], 0.0)
    out[...] = jnp.dot(h, w3[...], preferred_element_type=jnp.float32) + b3[...]


def _mlp_tc(xu, xi, w1a, w1b, b1, w2, b2, w3, b3, blk=2048):
    B, D = xu.shape
    H1 = w1a.shape[1]
    H2 = w2.shape[1]
    grid = (B // blk,)
    full = lambda shape: pl.BlockSpec(shape, lambda i: (0, 0))
    return pl.pallas_call(
        _mlp_body,
        grid=grid,
        in_specs=[
            pl.BlockSpec((blk, D), lambda i: (i, 0)),
            pl.BlockSpec((blk, D), lambda i: (i, 0)),
            full((D, H1)), full((D, H1)), full((1, H1)),
            full((H1, H2)), full((1, H2)),
            full((H2, 1)), full((1, 1)),
        ],
        out_specs=pl.BlockSpec((blk, 1), lambda i: (i, 0)),
        out_shape=jax.ShapeDtypeStruct((B, 1), jnp.float32),
    )(xu, xi, w1a, w1b, b1, w2, b2, w3, b3)


def kernel(user_ids, item_ids, user_table, item_table, W1, b1, W2, b2, W3, b3):
    B = user_ids.shape[0]
    D = user_table.shape[1]
    uids = user_ids.astype(jnp.int32)
    iids = item_ids.astype(jnp.int32)
    u_emb, i_emb = _make_sc_gather(B, D)(uids, iids, user_table, item_table)
    out = _mlp_tc(u_emb, i_emb, W1[:D], W1[D:], b1.reshape(1, -1),
                  W2, b2.reshape(1, -1), W3, b3.reshape(1, -1))
    return jnp.squeeze(out, axis=1)


# speedup vs baseline: 3.6329x; 2.4301x over previous
import functools

import jax
import jax.numpy as jnp
from jax import lax
from jax.experimental import pallas as pl
from jax.experimental.pallas import tpu as pltpu
from jax.experimental.pallas import tpu_sc as plsc

_INFO = plsc.get_sparse_core_info()
_NC, _NS = _INFO.num_cores, _INFO.num_subcores
_NW = _NC * _NS


def _make_sc_gather(B, D):
    bpw = B // _NW
    mesh = plsc.VectorSubcoreMesh(core_axis_name="c", subcore_axis_name="s")

    G = 16
    W = 128

    @functools.partial(
        pl.kernel,
        mesh=mesh,
        out_type=[
            jax.ShapeDtypeStruct((D, B), jnp.float32),
            jax.ShapeDtypeStruct((D, B), jnp.float32),
        ],
        scratch_types=[
            pltpu.VMEM((2, bpw), jnp.int32),
            pltpu.VMEM((D, G * W), jnp.float32),
            pltpu.VMEM((D, bpw), jnp.float32),
            pltpu.VMEM((D, bpw), jnp.float32),
            pltpu.SemaphoreType.DMA,
        ],
        compiler_params=pltpu.CompilerParams(
            use_tc_tiling_on_sc=True, needs_layout_passes=False),
    )
    def gather(uids, iids, utabT, itabT, u_outT, i_outT,
               ids_vm, slab, uoutT, ioutT, sem):
        wid = lax.axis_index("s") * _NC + lax.axis_index("c")
        base = wid * bpw
        pltpu.sync_copy(uids.at[pl.ds(base, bpw)], ids_vm.at[0])
        pltpu.sync_copy(iids.at[pl.ds(base, bpw)], ids_vm.at[1])
        lane = lax.iota(jnp.int32, G)

        def make_issue(t, tabT, outT):
            def issue(g, _):
                vec = ids_vm[t, pl.ds(g * G, G)]
                for l in range(G):
                    tid = jnp.sum(jnp.where(lane == l, vec, 0))
                    off = pl.multiple_of(tid & ~(W - 1), W)
                    pltpu.make_async_copy(
                        tabT.at[:, pl.ds(off, W)],
                        slab.at[:, pl.ds(l * W, W)], sem).start()
                pltpu.make_async_copy(
                    tabT.at[:, pl.ds(0, G * W)], slab, sem).wait()
                idx1 = lane * W + (vec & (W - 1))
                for c in range(D):
                    idx0 = jnp.broadcast_to(jnp.int32(c), (G,))
                    row = plsc.load_gather(slab, [idx0, idx1])
                    outT[c, pl.ds(g * G, G)] = row
                return 0

            lax.fori_loop(0, bpw // G, issue, 0)

        make_issue(0, utabT, uoutT)
        make_issue(1, itabT, ioutT)
        pltpu.sync_copy(uoutT, u_outT.at[:, pl.ds(base, bpw)])
        pltpu.sync_copy(ioutT, i_outT.at[:, pl.ds(base, bpw)])

    return gather


def _mlp_body(xuT, xiT, w1aT, w1bT, b1c, w2T, b2c, w3T, b3c, outT):
    h = jnp.dot(w1aT[...], xuT[...], preferred_element_type=jnp.float32)
    h = h + jnp.dot(w1bT[...], xiT[...], preferred_element_type=jnp.float32)
    h = jnp.maximum(h + b1c[...], 0.0)
    h = jnp.maximum(
        jnp.dot(w2T[...], h, preferred_element_type=jnp.float32) + b2c[...],
        0.0)
    outT[...] = (
        jnp.dot(w3T[...], h, preferred_element_type=jnp.float32) + b3c[...])


def _mlp_tc(xuT, xiT, w1aT, w1bT, b1c, w2T, b2c, w3T, b3c, blk=2048):
    D, B = xuT.shape
    H1 = w1aT.shape[0]
    H2 = w2T.shape[0]
    grid = (B // blk,)
    full = lambda shape: pl.BlockSpec(shape, lambda i: (0, 0))
    return pl.pallas_call(
        _mlp_body,
        grid=grid,
        in_specs=[
            pl.BlockSpec((D, blk), lambda i: (0, i)),
            pl.BlockSpec((D, blk), lambda i: (0, i)),
            full((H1, D)), full((H1, D)), full((H1, 1)),
            full((H2, H1)), full((H2, 1)),
            full((1, H2)), full((1, 1)),
        ],
        out_specs=pl.BlockSpec((1, blk), lambda i: (0, i)),
        out_shape=jax.ShapeDtypeStruct((1, B), jnp.float32),
    )(xuT, xiT, w1aT, w1bT, b1c, w2T, b2c, w3T, b3c)


def kernel(user_ids, item_ids, user_table, item_table, W1, b1, W2, b2, W3, b3):
    B = user_ids.shape[0]
    D = user_table.shape[1]
    uids = user_ids.astype(jnp.int32)
    iids = item_ids.astype(jnp.int32)
    u_embT, i_embT = _make_sc_gather(B, D)(
        uids, iids, user_table.T, item_table.T)
    W1T = W1.T
    outT = _mlp_tc(u_embT, i_embT, W1T[:, :D], W1T[:, D:], b1.reshape(-1, 1),
                   W2.T, b2.reshape(-1, 1), W3.T, b3.reshape(1, 1))
    return outT.reshape(B)
